# baseline (speedup 1.0000x reference)
import functools

import jax
import jax.numpy as jnp
from jax.experimental import pallas as pl

_NB = 16


def _reduce_body(x_ref, o_ref, *, scale):
    o_ref[...] = jnp.sum(x_ref[...], axis=1, keepdims=True) * scale


def kernel(x, protos):
    N, M, C, T, V = x.shape
    P = protos.shape[0]
    scale = 1.0 / (M * P)
    rows = M * V * T

    xt = jnp.transpose(x, (0, 1, 4, 3, 2)).reshape(N, rows, C)

    out = pl.pallas_call(
        functools.partial(_reduce_body, scale=scale),
        grid=(N // _NB,),
        in_specs=[pl.BlockSpec((_NB, rows, C), lambda i: (i, 0, 0))],
        out_specs=pl.BlockSpec((_NB, 1, C), lambda i: (i, 0, 0)),
        out_shape=jax.ShapeDtypeStruct((N, 1, C), x.dtype),
    )(xt)
    return out.reshape(N, C)
